# Initial kernel scaffold; baseline (speedup 1.0000x reference)
import jax
import jax.numpy as jnp
from jax.experimental import pallas as pl


def kernel(edge_index, edge_weight, user_emb, item_emb, W_h1, W_3, W_4):
    raise NotImplementedError("write your pallas kernel here")


# speedup vs baseline: 3.0988x; 3.0988x over previous
import functools

import jax
import jax.numpy as jnp
from jax import lax
from jax.experimental import pallas as pl
from jax.experimental.pallas import tpu as pltpu
from jax.experimental.pallas import tpu_sc as plsc

NUM_USER = 5000
NUM_ITEM = 5000
DIM = 128
N_NODES = NUM_USER + NUM_ITEM
N_EDGES = 320000

NC = 2
NS = 16
NW = NC * NS
LANES = 16

CHUNK = 128
K_PER_W = 80
E_PAD = NW * K_PER_W * CHUNK
N_PAD = 10240
ROWS_PER_TILE = N_PAD // NS
BLK = 128


def _spmm_body(x_hbm, col_hbm, row_hbm, w_hbm, out_hbm,
               acc, col_v, row_v, w_v, rows_v):
    c = lax.axis_index("c")
    s = lax.axis_index("s")
    wid = s * NC + c

    def _zero_buf(i, _):
        for j in range(DIM // LANES):
            rows_v[i, pl.ds(LANES * j, LANES)] = jnp.zeros((LANES,), jnp.float32)
        return 0
    lax.fori_loop(0, BLK, _zero_buf, 0)

    def _zero_acc(k, _):
        pltpu.sync_copy(rows_v, acc.at[pl.ds(s * ROWS_PER_TILE + k * BLK, BLK)])
        return 0
    lax.fori_loop(0, ROWS_PER_TILE // BLK, _zero_acc, 0)
    plsc.subcore_barrier()

    base = wid * K_PER_W
    pltpu.sync_copy(col_hbm.at[pl.ds(base, K_PER_W)], col_v)
    pltpu.sync_copy(row_hbm.at[pl.ds(base, K_PER_W)], row_v)
    pltpu.sync_copy(w_hbm.at[pl.ds(base * CHUNK, K_PER_W * CHUNK)], w_v)

    def _chunk(k, _):
        pltpu.sync_copy(x_hbm.at[col_v.at[k]], rows_v)

        def _scale_group(g, _):
            wvec = w_v[pl.ds(k * CHUNK + g * LANES, LANES)]

            def _scale_row(i, _):
                r = g * LANES + i
                wb = wvec.at[jnp.full((LANES,), i, jnp.int32)].get(
                    mode="promise_in_bounds")
                for j in range(DIM // LANES):
                    sl = (r, pl.ds(LANES * j, LANES))
                    rows_v[sl] = rows_v[sl] * wb
                return 0
            lax.fori_loop(0, LANES, _scale_row, 0)
            return 0
        lax.fori_loop(0, CHUNK // LANES, _scale_group, 0)

        pltpu.sync_copy(rows_v, acc.at[row_v.at[k]], add=True)
        return 0
    lax.fori_loop(0, K_PER_W, _chunk, 0)
    plsc.subcore_barrier()

    def _writeout(k, _):
        r0 = s * ROWS_PER_TILE + k * BLK
        pltpu.sync_copy(acc.at[pl.ds(r0, BLK)], rows_v)
        pltpu.sync_copy(rows_v, out_hbm.at[c, pl.ds(r0, BLK)])
        return 0
    lax.fori_loop(0, ROWS_PER_TILE // BLK, _writeout, 0)


def _spmm_sc(x, col2d, row2d, w2d):
    mesh = plsc.VectorSubcoreMesh(core_axis_name="c", subcore_axis_name="s")
    return pl.kernel(
        _spmm_body,
        out_type=jax.ShapeDtypeStruct((NC, N_PAD, DIM), jnp.float32),
        mesh=mesh,
        scratch_types=[
            pltpu.VMEM_SHARED((N_PAD, DIM), jnp.float32),
            pltpu.VMEM((K_PER_W, CHUNK), jnp.int32),
            pltpu.VMEM((K_PER_W, CHUNK), jnp.int32),
            pltpu.VMEM((K_PER_W * CHUNK,), jnp.float32),
            pltpu.VMEM((CHUNK, DIM), jnp.float32),
        ],
    )(x, col2d, row2d, w2d)


def _lrelu(v):
    return jnp.where(v > 0, v, 0.1 * v)


def _dense_body(p_ref, w1_ref, w3_ref, w4_ref, o_ref):
    p = p_ref[0] + p_ref[1]
    y = _lrelu(p)
    nt = (((1,), (1,)), ((), ()))
    h = lax.dot_general(y, w1_ref[...], nt, preferred_element_type=jnp.float32)
    g = _lrelu(lax.dot_general(h, w3_ref[...], nt,
                               preferred_element_type=jnp.float32))
    o_ref[...] = lax.dot_general(g, w4_ref[...], nt,
                                 preferred_element_type=jnp.float32)


def _dense_tc(partials, W_h1, W_3, W_4):
    nblk = 8
    rb = N_PAD // nblk
    return pl.pallas_call(
        _dense_body,
        grid=(nblk,),
        in_specs=[
            pl.BlockSpec((NC, rb, DIM), lambda i: (0, i, 0)),
            pl.BlockSpec((DIM, DIM), lambda i: (0, 0)),
            pl.BlockSpec((DIM, DIM), lambda i: (0, 0)),
            pl.BlockSpec((DIM, DIM), lambda i: (0, 0)),
        ],
        out_specs=pl.BlockSpec((rb, DIM), lambda i: (i, 0)),
        out_shape=jax.ShapeDtypeStruct((N_PAD, DIM), jnp.float32),
    )(partials, W_h1, W_3, W_4)


def kernel(edge_index, edge_weight, user_emb, item_emb, W_h1, W_3, W_4):
    x0 = jnp.concatenate([user_emb, item_emb], axis=0)
    x0p = jnp.concatenate(
        [x0, jnp.zeros((N_PAD - N_NODES, DIM), jnp.float32)], axis=0)

    pad = E_PAD - N_EDGES
    row = jnp.concatenate([edge_index[0], jnp.zeros((pad,), jnp.int32)])
    col = jnp.concatenate([edge_index[1], jnp.zeros((pad,), jnp.int32)])
    w = jnp.concatenate([edge_weight, jnp.zeros((pad,), jnp.float32)])
    row2d = row.reshape(-1, CHUNK)
    col2d = col.reshape(-1, CHUNK)

    p1 = _spmm_sc(x0p, col2d, row2d, w)
    x1p = _dense_tc(p1, W_h1, W_3, W_4)
    p2 = _spmm_sc(x1p, col2d, row2d, w)
    x2p = _dense_tc(p2, W_h1, W_3, W_4)

    x1 = x1p[:N_NODES]
    x2 = x2p[:N_NODES]
    return (x0, x1, x2, user_emb, item_emb, W_h1, W_3, W_4)


# speedup vs baseline: 3.4482x; 1.1128x over previous
import jax
import jax.numpy as jnp
from jax import lax
from jax.experimental import pallas as pl
from jax.experimental.pallas import tpu as pltpu
from jax.experimental.pallas import tpu_sc as plsc

NUM_USER = 5000
NUM_ITEM = 5000
DIM = 128
N_NODES = NUM_USER + NUM_ITEM
N_EDGES = 320000

NC = 2
NS = 16
NW = NC * NS
LANES = 16

CHUNK = 80
K_PER_W = 128
EDGES_PER_W = CHUNK * K_PER_W
E_PAD = NW * EDGES_PER_W
N_PAD = 10240
ROWS_PER_TILE = N_PAD // NS


def _scale_chunk(gbuf, wbuf):
    def _group(g, _):
        wvec = wbuf[pl.ds(g * LANES, LANES)]
        for i in range(LANES):
            wb = wvec.at[jnp.full((LANES,), i, jnp.int32)].get(
                mode="promise_in_bounds")
            r = g * LANES + i
            for j in range(DIM // LANES):
                sl = (r, pl.ds(LANES * j, LANES))
                gbuf[sl] = gbuf[sl] * wb
        return 0
    lax.fori_loop(0, CHUNK // LANES, _group, 0)


def _spmm_body(x_hbm, col_hbm, row_hbm, w_hbm, out_hbm, acc, *rest):
    gb = rest[0:4]
    cb = rest[4:12]
    rb = rest[12:20]
    wb = rest[20:28]
    gsem = rest[28:32]
    ssem = rest[32:36]
    isem = rest[36:44]

    c = lax.axis_index("c")
    s = lax.axis_index("s")
    wid = s * NC + c
    base = wid * EDGES_PER_W

    def _issue_idx(eoff, t):
        pltpu.async_copy(col_hbm.at[pl.ds(eoff, CHUNK)], cb[t], isem[t])
        pltpu.async_copy(row_hbm.at[pl.ds(eoff, CHUNK)], rb[t], isem[t])
        pltpu.async_copy(w_hbm.at[pl.ds(eoff, CHUNK)], wb[t], isem[t])

    def _wait_idx(t):
        pltpu.make_async_copy(col_hbm.at[pl.ds(0, CHUNK)], cb[t], isem[t]).wait()
        pltpu.make_async_copy(row_hbm.at[pl.ds(0, CHUNK)], rb[t], isem[t]).wait()
        pltpu.make_async_copy(w_hbm.at[pl.ds(0, CHUNK)], wb[t], isem[t]).wait()

    def _issue_gather(t, p):
        pltpu.async_copy(x_hbm.at[cb[t]], gb[p], gsem[p])

    def _wait_gather(p):
        pltpu.make_async_copy(x_hbm.at[cb[0]], gb[p], gsem[p]).wait()

    def _issue_scatter(p, t):
        pltpu.async_copy(gb[p], acc.at[rb[t]], ssem[p], add=True)

    def _wait_scatter(p):
        pltpu.make_async_copy(gb[p], acc.at[rb[0]], ssem[p]).wait()

    def _zero_buf(i, _):
        for j in range(DIM // LANES):
            gb[0][i, pl.ds(LANES * j, LANES)] = jnp.zeros((LANES,), jnp.float32)
        return 0
    lax.fori_loop(0, CHUNK, _zero_buf, 0)

    def _zero_acc(k, _):
        pltpu.sync_copy(gb[0],
                        acc.at[pl.ds(s * ROWS_PER_TILE + k * CHUNK, CHUNK)])
        return 0
    lax.fori_loop(0, ROWS_PER_TILE // CHUNK, _zero_acc, 0)
    plsc.subcore_barrier()

    for t in range(6):
        _issue_idx(base + t * CHUNK, t)
    _wait_idx(0)
    _wait_idx(1)
    _issue_gather(0, 0)
    _issue_gather(1, 1)

    def _octet(k8, _):
        for u in range(8):
            k = 8 * k8 + u
            p = u % 4
            p2 = (u + 2) % 4
            q2 = (u + 2) % 8
            q6 = (u + 6) % 8
            if u >= 2:
                _wait_scatter(p2)
            else:
                @pl.when(k8 > 0)
                def _():
                    _wait_scatter(p2)
            _wait_idx(q2)
            _issue_gather(q2, p2)
            kp6 = jnp.minimum(k + 6, K_PER_W - 1)
            _issue_idx(base + kp6 * CHUNK, q6)
            _wait_gather(p)
            _scale_chunk(gb[p], wb[u])
            _issue_scatter(p, u)
        return 0
    lax.fori_loop(0, K_PER_W // 8, _octet, 0)

    for t in (2, 3, 4, 5):
        _wait_idx(t)
    _wait_gather(0)
    _wait_gather(1)
    _wait_scatter(2)
    _wait_scatter(3)
    plsc.subcore_barrier()

    r0 = s * ROWS_PER_TILE
    pltpu.sync_copy(acc.at[pl.ds(r0, ROWS_PER_TILE)],
                    out_hbm.at[c, pl.ds(r0, ROWS_PER_TILE)])


def _spmm_sc(x, col, row, w):
    mesh = plsc.VectorSubcoreMesh(core_axis_name="c", subcore_axis_name="s")
    scratch = (
        [pltpu.VMEM_SHARED((N_PAD, DIM), jnp.float32)]
        + [pltpu.VMEM((CHUNK, DIM), jnp.float32)] * 4
        + [pltpu.VMEM((CHUNK,), jnp.int32)] * 8
        + [pltpu.VMEM((CHUNK,), jnp.int32)] * 8
        + [pltpu.VMEM((CHUNK,), jnp.float32)] * 8
        + [pltpu.SemaphoreType.DMA] * 16
    )
    return pl.kernel(
        _spmm_body,
        out_type=jax.ShapeDtypeStruct((NC, N_PAD, DIM), jnp.float32),
        mesh=mesh,
        scratch_types=scratch,
    )(x, col, row, w)


def _lrelu(v):
    return jnp.where(v > 0, v, 0.1 * v)


def _dense_body(p_ref, w1_ref, w3_ref, w4_ref, o_ref):
    p = p_ref[0] + p_ref[1]
    y = _lrelu(p)
    nt = (((1,), (1,)), ((), ()))
    h = lax.dot_general(y, w1_ref[...], nt, preferred_element_type=jnp.float32)
    g = _lrelu(lax.dot_general(h, w3_ref[...], nt,
                               preferred_element_type=jnp.float32))
    o_ref[...] = lax.dot_general(g, w4_ref[...], nt,
                                 preferred_element_type=jnp.float32)


def _dense_tc(partials, W_h1, W_3, W_4):
    nblk = 8
    rb = N_PAD // nblk
    return pl.pallas_call(
        _dense_body,
        grid=(nblk,),
        in_specs=[
            pl.BlockSpec((NC, rb, DIM), lambda i: (0, i, 0)),
            pl.BlockSpec((DIM, DIM), lambda i: (0, 0)),
            pl.BlockSpec((DIM, DIM), lambda i: (0, 0)),
            pl.BlockSpec((DIM, DIM), lambda i: (0, 0)),
        ],
        out_specs=pl.BlockSpec((rb, DIM), lambda i: (i, 0)),
        out_shape=jax.ShapeDtypeStruct((N_NODES, DIM), jnp.float32),
    )(partials, W_h1, W_3, W_4)


def kernel(edge_index, edge_weight, user_emb, item_emb, W_h1, W_3, W_4):
    x0 = jnp.concatenate([user_emb, item_emb], axis=0)

    pad = E_PAD - N_EDGES
    row = jnp.concatenate([edge_index[0], jnp.zeros((pad,), jnp.int32)])
    col = jnp.concatenate([edge_index[1], jnp.zeros((pad,), jnp.int32)])
    w = jnp.concatenate([edge_weight, jnp.zeros((pad,), jnp.float32)])

    p1 = _spmm_sc(x0, col, row, w)
    x1 = _dense_tc(p1, W_h1, W_3, W_4)
    p2 = _spmm_sc(x1, col, row, w)
    x2 = _dense_tc(p2, W_h1, W_3, W_4)

    return (x0, x1, x2, user_emb, item_emb, W_h1, W_3, W_4)


# speedup vs baseline: 3.6389x; 1.0553x over previous
import jax
import jax.numpy as jnp
from jax import lax
from jax.experimental import pallas as pl
from jax.experimental.pallas import tpu as pltpu
from jax.experimental.pallas import tpu_sc as plsc

NUM_USER = 5000
NUM_ITEM = 5000
DIM = 128
N_NODES = NUM_USER + NUM_ITEM
N_EDGES = 320000

NC = 2
NS = 16
NW = NC * NS
LANES = 16

CHUNK = 32
K_PER_W = 320
EDGES_PER_W = CHUNK * K_PER_W
E_PAD = NW * EDGES_PER_W
N_PAD = 10240
ROWS_PER_TILE = N_PAD // NS

NGB = 8
NIX = 8
GDEPTH = 4
IDEPTH = 6


def _scale_chunk(gbuf, wbuf):
    def _group(g, _):
        wvec = wbuf[pl.ds(g * LANES, LANES)]

        def _rows(i4, _):
            for di in range(4):
                i = i4 * 4 + di
                wb = wvec.at[jnp.full((LANES,), i, jnp.int32)].get(
                    mode="promise_in_bounds")
                r = g * LANES + i
                for j in range(DIM // LANES):
                    sl = (r, pl.ds(LANES * j, LANES))
                    gbuf[sl] = gbuf[sl] * wb
            return 0
        lax.fori_loop(0, LANES // 4, _rows, 0)
        return 0
    lax.fori_loop(0, CHUNK // LANES, _group, 0)


def _spmm_body(x_hbm, col_hbm, row_hbm, w_hbm, out_hbm, acc, *rest):
    gb = rest[0:NGB]
    cb = rest[NGB:NGB + NIX]
    rb = rest[NGB + NIX:NGB + 2 * NIX]
    wb = rest[NGB + 2 * NIX:NGB + 3 * NIX]
    gsem = rest[NGB + 3 * NIX:2 * NGB + 3 * NIX]
    ssem = rest[2 * NGB + 3 * NIX:3 * NGB + 3 * NIX]
    isem = rest[3 * NGB + 3 * NIX:3 * NGB + 4 * NIX]

    c = lax.axis_index("c")
    s = lax.axis_index("s")
    wid = s * NC + c
    base = wid * EDGES_PER_W

    def _issue_idx(k, t):
        eoff = base + k * CHUNK
        pltpu.async_copy(col_hbm.at[pl.ds(eoff, CHUNK)], cb[t], isem[t])
        pltpu.async_copy(row_hbm.at[pl.ds(eoff, CHUNK)], rb[t], isem[t])
        pltpu.async_copy(w_hbm.at[pl.ds(eoff, CHUNK)], wb[t], isem[t])

    def _wait_idx(t):
        pltpu.make_async_copy(col_hbm.at[pl.ds(0, CHUNK)], cb[t], isem[t]).wait()
        pltpu.make_async_copy(row_hbm.at[pl.ds(0, CHUNK)], rb[t], isem[t]).wait()
        pltpu.make_async_copy(w_hbm.at[pl.ds(0, CHUNK)], wb[t], isem[t]).wait()

    def _issue_gather(t, p):
        pltpu.async_copy(x_hbm.at[cb[t]], gb[p], gsem[p])

    def _wait_gather(p):
        pltpu.make_async_copy(x_hbm.at[cb[0]], gb[p], gsem[p]).wait()

    def _issue_scatter(p, t):
        pltpu.async_copy(gb[p], acc.at[rb[t]], ssem[p], add=True)

    def _wait_scatter(p):
        pltpu.make_async_copy(gb[p], acc.at[rb[0]], ssem[p]).wait()

    def _zero_buf(i, _):
        for j in range(DIM // LANES):
            gb[0][i, pl.ds(LANES * j, LANES)] = jnp.zeros((LANES,), jnp.float32)
        return 0
    lax.fori_loop(0, CHUNK, _zero_buf, 0)

    def _zero_acc(k, _):
        pltpu.sync_copy(gb[0],
                        acc.at[pl.ds(s * ROWS_PER_TILE + k * CHUNK, CHUNK)])
        return 0
    lax.fori_loop(0, ROWS_PER_TILE // CHUNK, _zero_acc, 0)
    plsc.subcore_barrier()

    for t in range(IDEPTH):
        _issue_idx(t, t)
    for t in range(GDEPTH):
        _wait_idx(t)
        _issue_gather(t, t)

    def _octet(k8, _):
        for u in range(NIX):
            k = NIX * k8 + u
            p = u % NGB
            pg = (u + GDEPTH) % NGB
            qg = (u + GDEPTH) % NIX
            qi = (u + IDEPTH) % NIX
            if u >= 2:
                _wait_scatter((u - 2) % NGB)
            else:
                @pl.when(k8 > 0)
                def _():
                    _wait_scatter((u - 2) % NGB)
            _wait_idx(qg)
            _issue_gather(qg, pg)
            kpi = jnp.minimum(k + IDEPTH, K_PER_W - 1)
            _issue_idx(kpi, qi)
            _wait_gather(p)
            _scale_chunk(gb[p], wb[u])
            _issue_scatter(p, u)
        return 0
    lax.fori_loop(0, K_PER_W // NIX, _octet, 0)

    for t in range(GDEPTH):
        _wait_gather(t % NGB)
    for t in range(GDEPTH, IDEPTH):
        _wait_idx(t % NIX)
    _wait_scatter((K_PER_W - 2) % NGB)
    _wait_scatter((K_PER_W - 1) % NGB)
    plsc.subcore_barrier()

    r0 = s * ROWS_PER_TILE
    pltpu.sync_copy(acc.at[pl.ds(r0, ROWS_PER_TILE)],
                    out_hbm.at[c, pl.ds(r0, ROWS_PER_TILE)])


def _spmm_sc(x, col, row, w):
    mesh = plsc.VectorSubcoreMesh(core_axis_name="c", subcore_axis_name="s")
    scratch = (
        [pltpu.VMEM_SHARED((N_PAD, DIM), jnp.float32)]
        + [pltpu.VMEM((CHUNK, DIM), jnp.float32)] * NGB
        + [pltpu.VMEM((CHUNK,), jnp.int32)] * NIX
        + [pltpu.VMEM((CHUNK,), jnp.int32)] * NIX
        + [pltpu.VMEM((CHUNK,), jnp.float32)] * NIX
        + [pltpu.SemaphoreType.DMA] * (2 * NGB + NIX)
    )
    return pl.kernel(
        _spmm_body,
        out_type=jax.ShapeDtypeStruct((NC, N_PAD, DIM), jnp.float32),
        mesh=mesh,
        scratch_types=scratch,
    )(x, col, row, w)


def _lrelu(v):
    return jnp.where(v > 0, v, 0.1 * v)


def _dense_body(p_ref, w1_ref, w3_ref, w4_ref, o_ref):
    p = p_ref[0] + p_ref[1]
    y = _lrelu(p)
    nt = (((1,), (1,)), ((), ()))
    h = lax.dot_general(y, w1_ref[...], nt, preferred_element_type=jnp.float32)
    g = _lrelu(lax.dot_general(h, w3_ref[...], nt,
                               preferred_element_type=jnp.float32))
    o_ref[...] = lax.dot_general(g, w4_ref[...], nt,
                                 preferred_element_type=jnp.float32)


def _dense_tc(partials, W_h1, W_3, W_4):
    nblk = 8
    rb = N_PAD // nblk
    return pl.pallas_call(
        _dense_body,
        grid=(nblk,),
        in_specs=[
            pl.BlockSpec((NC, rb, DIM), lambda i: (0, i, 0)),
            pl.BlockSpec((DIM, DIM), lambda i: (0, 0)),
            pl.BlockSpec((DIM, DIM), lambda i: (0, 0)),
            pl.BlockSpec((DIM, DIM), lambda i: (0, 0)),
        ],
        out_specs=pl.BlockSpec((rb, DIM), lambda i: (i, 0)),
        out_shape=jax.ShapeDtypeStruct((N_NODES, DIM), jnp.float32),
    )(partials, W_h1, W_3, W_4)


def kernel(edge_index, edge_weight, user_emb, item_emb, W_h1, W_3, W_4):
    x0 = jnp.concatenate([user_emb, item_emb], axis=0)

    pad = E_PAD - N_EDGES
    row = jnp.concatenate([edge_index[0], jnp.zeros((pad,), jnp.int32)])
    col = jnp.concatenate([edge_index[1], jnp.zeros((pad,), jnp.int32)])
    w = jnp.concatenate([edge_weight, jnp.zeros((pad,), jnp.float32)])

    p1 = _spmm_sc(x0, col, row, w)
    x1 = _dense_tc(p1, W_h1, W_3, W_4)
    p2 = _spmm_sc(x1, col, row, w)
    x2 = _dense_tc(p2, W_h1, W_3, W_4)

    return (x0, x1, x2, user_emb, item_emb, W_h1, W_3, W_4)
